# baseline (speedup 1.0000x reference)
import functools

import jax
import jax.numpy as jnp
from jax import lax
from jax.experimental import pallas as pl
from jax.experimental.pallas import tpu as pltpu
from jax.experimental.pallas import tpu_sc as plsc

NUM_EMBEDDINGS = 1000000
EMBEDDING_DIM = 64
BATCH = 4096
HIST_LEN = 200

CHUNK = 128
N_ROWS = BATCH * HIST_LEN
N_CHUNKS = N_ROWS // CHUNK

NBUF = 8
PREF = 4


def _make_sc_gather():
    info = plsc.get_sparse_core_info()
    nw = info.num_cores * info.num_subcores
    chunks_per_w = N_CHUNKS // nw
    assert chunks_per_w % NBUF == 0
    groups = chunks_per_w // NBUF

    mesh = plsc.VectorSubcoreMesh(core_axis_name="c", subcore_axis_name="s")

    @functools.partial(
        pl.kernel,
        mesh=mesh,
        out_type=jax.ShapeDtypeStruct((N_ROWS, 2 * EMBEDDING_DIM), jnp.float32),
        scratch_types=[
            pltpu.VMEM((chunks_per_w, CHUNK), jnp.int32),
            pltpu.VMEM((NBUF, CHUNK, EMBEDDING_DIM), jnp.float32),
            pltpu.SemaphoreType.DMA((NBUF,)),
            pltpu.SemaphoreType.DMA((NBUF,)),
        ],
        compiler_params=pltpu.CompilerParams(use_tc_tiling_on_sc=False),
    )
    def gather_kernel(idx_hbm, table_hbm, out_hbm, idx_v, bufs, gsem, ssem):
        wid = lax.axis_index("s") * info.num_cores + lax.axis_index("c")
        chunk_base = wid * chunks_per_w
        pltpu.sync_copy(idx_hbm.at[pl.ds(chunk_base, chunks_per_w)], idx_v)

        def gather(j, b):
            pltpu.make_async_copy(
                table_hbm.at[idx_v.at[j]], bufs.at[b], gsem.at[b]
            ).start()

        def store(j, b):
            pltpu.make_async_copy(
                bufs.at[b],
                out_hbm.at[
                    pl.ds((chunk_base + j) * CHUNK, CHUNK),
                    pl.ds(0, EMBEDDING_DIM),
                ],
                ssem.at[b],
            ).start()

        def wait_gather(b):
            pltpu.make_async_copy(
                table_hbm.at[idx_v.at[0]], bufs.at[b], gsem.at[b]
            ).wait()

        def wait_store(b):
            pltpu.make_async_copy(
                bufs.at[b],
                out_hbm.at[pl.ds(0, CHUNK), pl.ds(0, EMBEDDING_DIM)],
                ssem.at[b],
            ).wait()

        for b in range(PREF):
            gather(b, b)

        def group(g, carry):
            for b in range(NBUF):
                j = g * NBUF + b
                jp = j + PREF
                bp = (b + PREF) % NBUF

                @pl.when(jp < chunks_per_w)
                def _():
                    @pl.when(jp >= NBUF)
                    def _():
                        wait_store(bp)

                    gather(jp, bp)

                wait_gather(b)
                store(j, b)
            return carry

        lax.fori_loop(0, groups, group, 0)
        for b in range(NBUF):
            wait_store(b)

    return gather_kernel


_gather = _make_sc_gather()


def kernel(token_ids, weight):
    w2 = jnp.pad(weight, ((0, 0), (0, 128 - EMBEDDING_DIM))).reshape(
        2 * NUM_EMBEDDINGS, EMBEDDING_DIM
    )
    idx2 = token_ids.reshape(N_CHUNKS, CHUNK).astype(jnp.int32) * 2
    out_pad = _gather(idx2, w2)
    return out_pad[:, :EMBEDDING_DIM].reshape(BATCH, HIST_LEN, EMBEDDING_DIM)
